# baseline (speedup 1.0000x reference)
import math

import jax
import jax.numpy as jnp
from jax import lax
from jax.experimental import pallas as pl
from jax.experimental.pallas import tpu as pltpu
from jax.experimental.pallas import tpu_sc as plsc

_B = 8
_N = 16
_T = 64
_C = 512
_EMB = 256
_L = _N * _T

_SC_CORES = 2
_SC_SUBCORES = 16
_NW = _SC_CORES * _SC_SUBCORES
_CHUNK = _B * _L // _NW


def _table_body(out_ref):
    t_f = lax.broadcasted_iota(jnp.int32, (_T, _EMB), 0).astype(jnp.float32)
    d_i = lax.broadcasted_iota(jnp.int32, (_T, _EMB), 1)
    inv_freq = jnp.exp((d_i // 4).astype(jnp.float32) *
                       (-math.log(10000.0) / (_EMB // 4)))
    ang = t_f * inv_freq
    out_ref[...] = jnp.where((d_i % 2) == 0, jnp.sin(ang), jnp.cos(ang))


def _main_body(lens_ref, feat_ref, w_ref, bias_ref,
               proj_ref, mask_ref, att_ref, acc_ref):
    b_idx = pl.program_id(0)

    acc_ref[...] = lax.dot_general(
        feat_ref[...], w_ref[...],
        dimension_numbers=(((1,), (1,)), ((), ())),
        preferred_element_type=jnp.float32) + bias_ref[...]

    starts = []
    s = 0
    for n in range(_N):
        starts.append(s)
        s = s + lens_ref[b_idx, n]
    total = s

    li_col = lax.broadcasted_iota(jnp.int32, (_L, 1), 0)
    li_row = lax.broadcasted_iota(jnp.int32, (1, _L), 1)
    cnt_c = jnp.zeros((_L, 1), jnp.int32)
    cnt_r = jnp.zeros((1, _L), jnp.int32)
    spos_c = jnp.zeros((_L, 1), jnp.int32)
    for n in range(_N):
        sel_c = li_col >= starts[n]
        cnt_c += sel_c.astype(jnp.int32)
        cnt_r += (li_row >= starts[n]).astype(jnp.int32)
        if n:
            spos_c = jnp.where(sel_c, starts[n], spos_c)
    valid_c = li_col < total
    valid_r = li_row < total
    cap_c = cnt_c - 1
    cap_r = cnt_r - 1

    src_c = jnp.where(valid_c, cap_c * _T + (li_col - spos_c), -1)
    perm = (li_row == src_c).astype(jnp.float32)
    proj_ref[...] = lax.dot_general(
        perm, acc_ref[...],
        dimension_numbers=(((1,), (0,)), ((), ())),
        preferred_element_type=jnp.float32)

    mask_ref[...] = jnp.logical_not(valid_r)
    cpr = jnp.where(valid_r, cap_r, _N + li_row)
    cpc = jnp.where(valid_c, cap_c, _N + li_col)
    att_ref[...] = cpc != cpr


def _sc_body(table_hbm, lens_hbm, pos_hbm, cap_hbm,
             lens_v, idx_v, cap_v, rows_v, sem):
    cid = lax.axis_index("c")
    sid = lax.axis_index("s")
    wid = sid * _SC_CORES + cid
    b = wid // 4
    l0 = (wid % 4) * _CHUNK

    pltpu.sync_copy(lens_hbm.at[b], lens_v)
    lens_vec = lens_v[...]

    starts = []
    s = 0
    for n in range(_N):
        starts.append(s)
        s = s + lens_vec[n]
    total = s

    iota16 = lax.iota(jnp.int32, 16)
    total_b = jnp.full((16,), total, jnp.int32)
    starts_b = [None] + [jnp.full((16,), starts[n], jnp.int32)
                         for n in range(1, _N)]
    for k in range(_CHUNK // 16):
        lvec = iota16 + jnp.full((16,), l0 + k * 16, jnp.int32)
        cnt = jnp.zeros((16,), jnp.int32)
        spos = jnp.zeros((16,), jnp.int32)
        for n in range(1, _N):
            sel = lvec >= starts_b[n]
            cnt = jnp.where(sel, cnt + 1, cnt)
            spos = jnp.where(sel, starts_b[n], spos)
        validv = lvec < total_b
        posv = jnp.where(validv, lvec - spos, 0)
        capv = jnp.where(validv, cnt, -1)
        idx_v[k // 8, pl.ds((k % 8) * 16, 16)] = posv
        cap_v[pl.ds(k * 16, 16)] = capv

    copies = [
        pltpu.async_copy(table_hbm.at[idx_v.at[j]],
                         rows_v.at[pl.ds(j * 128, 128)], sem)
        for j in range(_CHUNK // 128)
    ]
    for c in copies:
        c.wait()
    pltpu.sync_copy(rows_v, pos_hbm.at[b, pl.ds(l0, _CHUNK)])
    pltpu.sync_copy(cap_v, cap_hbm.at[b, pl.ds(l0, _CHUNK)])


def _run_table():
    return pl.pallas_call(
        _table_body,
        out_shape=jax.ShapeDtypeStruct((_T, _EMB), jnp.float32),
    )()


def _run_main(feat_flat, lens, w, bias2d):
    return pl.pallas_call(
        _main_body,
        grid=(_B,),
        in_specs=[
            pl.BlockSpec(memory_space=pltpu.SMEM),
            pl.BlockSpec((None, _L, _C), lambda b: (b, 0, 0)),
            pl.BlockSpec((_EMB, _C), lambda b: (0, 0)),
            pl.BlockSpec((1, _EMB), lambda b: (0, 0)),
        ],
        out_specs=[
            pl.BlockSpec((None, _L, _EMB), lambda b: (b, 0, 0)),
            pl.BlockSpec((None, 1, _L), lambda b: (b, 0, 0)),
            pl.BlockSpec((None, _L, _L), lambda b: (b, 0, 0)),
        ],
        out_shape=[
            jax.ShapeDtypeStruct((_B, _L, _EMB), jnp.float32),
            jax.ShapeDtypeStruct((_B, 1, _L), jnp.bool_),
            jax.ShapeDtypeStruct((_B, _L, _L), jnp.bool_),
        ],
        scratch_shapes=[pltpu.VMEM((_L, _EMB), jnp.float32)],
    )(lens, feat_flat, w, bias2d)


def _run_sc(table, lens):
    mesh = plsc.VectorSubcoreMesh(core_axis_name="c", subcore_axis_name="s")
    f = pl.kernel(
        _sc_body,
        out_type=[
            jax.ShapeDtypeStruct((_B, _L, _EMB), jnp.float32),
            jax.ShapeDtypeStruct((_B, _L), jnp.int32),
        ],
        mesh=mesh,
        scratch_types=[
            pltpu.VMEM((_N,), jnp.int32),
            pltpu.VMEM((_CHUNK // 128, 128), jnp.int32),
            pltpu.VMEM((_CHUNK,), jnp.int32),
            pltpu.VMEM((_CHUNK, _EMB), jnp.float32),
            pltpu.SemaphoreType.DMA,
        ],
    )
    return f(table, lens)


def kernel(batch_feat, batch_mask_lens, W, b):
    feat_flat = batch_feat.reshape(_B, _L, _C)
    bias2d = b.reshape(1, _EMB)
    table = _run_table()
    proj, mask3, att = _run_main(feat_flat, batch_mask_lens, W, bias2d)
    pos, cap = _run_sc(table, batch_mask_lens)
    return (proj, pos, mask3.reshape(_B, _L), att, cap)


# speedup vs baseline: 3.7463x; 3.7463x over previous
import math

import jax
import jax.numpy as jnp
from jax import lax
from jax.experimental import pallas as pl
from jax.experimental.pallas import tpu as pltpu
from jax.experimental.pallas import tpu_sc as plsc

_B = 8
_N = 16
_T = 64
_C = 512
_EMB = 256
_L = _N * _T

_SC_CORES = 2
_SC_SUBCORES = 16
_NW = _SC_CORES * _SC_SUBCORES
_CHUNK = _B * _L // _NW


def _table_body(out_ref):
    t_f = lax.broadcasted_iota(jnp.int32, (_T, _EMB), 0).astype(jnp.float32)
    d_i = lax.broadcasted_iota(jnp.int32, (_T, _EMB), 1)
    inv_freq = jnp.exp((d_i // 4).astype(jnp.float32) *
                       (-math.log(10000.0) / (_EMB // 4)))
    ang = t_f * inv_freq
    out_ref[...] = jnp.where((d_i % 2) == 0, jnp.sin(ang), jnp.cos(ang))


def _main_body(lens_ref, feat_ref, w_ref, bias_ref,
               proj_ref, pos_ref, mask_ref, att_ref, acc_ref):
    b_idx = pl.program_id(0)

    acc_ref[...] = lax.dot_general(
        feat_ref[...], w_ref[...],
        dimension_numbers=(((1,), (1,)), ((), ())),
        preferred_element_type=jnp.float32) + bias_ref[...]

    t_f = lax.broadcasted_iota(jnp.int32, (_T, _EMB), 0).astype(jnp.float32)
    d_i = lax.broadcasted_iota(jnp.int32, (_T, _EMB), 1)
    inv_freq = jnp.exp((d_i // 4).astype(jnp.float32) *
                       (-math.log(10000.0) / (_EMB // 4)))
    ang = t_f * inv_freq
    table = jnp.where((d_i % 2) == 0, jnp.sin(ang), jnp.cos(ang))

    starts = []
    s = 0
    for n in range(_N):
        starts.append(s)
        s = s + lens_ref[b_idx, n]
    total = s

    li_col = lax.broadcasted_iota(jnp.int32, (_L, 1), 0)
    li_row = lax.broadcasted_iota(jnp.int32, (1, _L), 1)
    cnt_c = jnp.zeros((_L, 1), jnp.int32)
    cnt_r = jnp.zeros((1, _L), jnp.int32)
    spos_c = jnp.zeros((_L, 1), jnp.int32)
    for n in range(_N):
        sel_c = li_col >= starts[n]
        cnt_c += sel_c.astype(jnp.int32)
        cnt_r += (li_row >= starts[n]).astype(jnp.int32)
        if n:
            spos_c = jnp.where(sel_c, starts[n], spos_c)
    valid_c = li_col < total
    valid_r = li_row < total
    cap_c = cnt_c - 1
    cap_r = cnt_r - 1

    src_c = jnp.where(valid_c, cap_c * _T + (li_col - spos_c), -1)
    perm = (li_row == src_c).astype(jnp.float32)
    proj_ref[...] = lax.dot_general(
        perm, acc_ref[...],
        dimension_numbers=(((1,), (0,)), ((), ())),
        preferred_element_type=jnp.float32)

    pos_id_c = jnp.where(valid_c, li_col - spos_c, 0)
    t_row = lax.broadcasted_iota(jnp.int32, (1, _T), 1)
    perm_t = (t_row == pos_id_c).astype(jnp.float32)
    pos_ref[...] = lax.dot_general(
        perm_t, table,
        dimension_numbers=(((1,), (0,)), ((), ())),
        preferred_element_type=jnp.float32)

    mask_ref[...] = jnp.logical_not(valid_r)
    cpr = jnp.where(valid_r, cap_r, _N + li_row)
    cpc = jnp.where(valid_c, cap_c, _N + li_col)
    att_ref[...] = cpc != cpr


def _sc_body(lens_hbm, cap_hbm, lens_v, cap_v):
    cid = lax.axis_index("c")
    sid = lax.axis_index("s")
    wid = sid * _SC_CORES + cid
    b = wid // 4
    l0 = (wid % 4) * _CHUNK

    pltpu.sync_copy(lens_hbm.at[b], lens_v)
    lens_vec = lens_v[...]

    starts = []
    s = 0
    for n in range(_N):
        starts.append(s)
        s = s + lens_vec[n]
    total = s

    iota16 = lax.iota(jnp.int32, 16)
    total_b = jnp.full((16,), total, jnp.int32)
    starts_b = [None] + [jnp.full((16,), starts[n], jnp.int32)
                         for n in range(1, _N)]
    for k in range(_CHUNK // 16):
        lvec = iota16 + jnp.full((16,), l0 + k * 16, jnp.int32)
        cnt = jnp.zeros((16,), jnp.int32)
        for n in range(1, _N):
            sel = lvec >= starts_b[n]
            cnt = jnp.where(sel, cnt + 1, cnt)
        validv = lvec < total_b
        capv = jnp.where(validv, cnt, -1)
        cap_v[pl.ds(k * 16, 16)] = capv

    pltpu.sync_copy(cap_v, cap_hbm.at[b, pl.ds(l0, _CHUNK)])


def _run_main(feat_flat, lens, w, bias2d):
    return pl.pallas_call(
        _main_body,
        grid=(_B,),
        in_specs=[
            pl.BlockSpec(memory_space=pltpu.SMEM),
            pl.BlockSpec((None, _L, _C), lambda b: (b, 0, 0)),
            pl.BlockSpec((_EMB, _C), lambda b: (0, 0)),
            pl.BlockSpec((1, _EMB), lambda b: (0, 0)),
        ],
        out_specs=[
            pl.BlockSpec((None, _L, _EMB), lambda b: (b, 0, 0)),
            pl.BlockSpec((None, _L, _EMB), lambda b: (b, 0, 0)),
            pl.BlockSpec((None, 1, _L), lambda b: (b, 0, 0)),
            pl.BlockSpec((None, _L, _L), lambda b: (b, 0, 0)),
        ],
        out_shape=[
            jax.ShapeDtypeStruct((_B, _L, _EMB), jnp.float32),
            jax.ShapeDtypeStruct((_B, _L, _EMB), jnp.float32),
            jax.ShapeDtypeStruct((_B, 1, _L), jnp.bool_),
            jax.ShapeDtypeStruct((_B, _L, _L), jnp.bool_),
        ],
        scratch_shapes=[pltpu.VMEM((_L, _EMB), jnp.float32)],
    )(lens, feat_flat, w, bias2d)


def _run_sc(lens):
    mesh = plsc.VectorSubcoreMesh(core_axis_name="c", subcore_axis_name="s")
    f = pl.kernel(
        _sc_body,
        out_type=[
            jax.ShapeDtypeStruct((_B, _L), jnp.int32),
        ],
        mesh=mesh,
        scratch_types=[
            pltpu.VMEM((_N,), jnp.int32),
            pltpu.VMEM((_CHUNK,), jnp.int32),
        ],
    )
    (cap,) = f(lens)
    return cap


def kernel(batch_feat, batch_mask_lens, W, b):
    feat_flat = batch_feat.reshape(_B, _L, _C)
    bias2d = b.reshape(1, _EMB)
    proj, pos, mask3, att = _run_main(feat_flat, batch_mask_lens, W, bias2d)
    cap = _run_sc(batch_mask_lens)
    return (proj, pos, mask3.reshape(_B, _L), att, cap)
